# baseline (speedup 1.0000x reference)
import jax
import jax.numpy as jnp
from jax.experimental import pallas as pl
from jax.experimental.pallas import tpu as pltpu

_N = 4096
_NHID = 512
_SUPPORT = 4
_NCLASS = 4
_PADC = 128

_TM = 256
_TK = 1024
_XW = _SUPPORT * _NHID


def _xw_body(x_ref, w_ref, out_ref):
    for r in range(_SUPPORT):
        out_ref[:, r * _NHID:(r + 1) * _NHID] = jnp.dot(
            x_ref[...], w_ref[r], preferred_element_type=jnp.float32
        ).astype(jnp.bfloat16)


def _layer1_body(adj_ref, xw_ref, w2_ref, adjb_ref, xw2_ref, acc_ref):
    a = adj_ref[...]
    xk = _TK * pl.program_id(1)
    part = None
    for r in range(_SUPPORT):
        ab = a[r].astype(jnp.bfloat16)
        adjb_ref[r] = ab
        p = jnp.dot(
            ab, xw_ref[pl.ds(xk, _TK), r * _NHID:(r + 1) * _NHID],
            preferred_element_type=jnp.float32,
        )
        part = p if part is None else part + p
    k = pl.program_id(1)

    @pl.when(k == 0)
    def _init():
        acc_ref[...] = part

    @pl.when(k > 0)
    def _accum():
        acc_ref[...] = acc_ref[...] + part

    @pl.when(k == pl.num_programs(1) - 1)
    def _epilogue():
        h1 = jnp.maximum(acc_ref[...], 0.0)
        for r in range(_SUPPORT):
            xw2_ref[:, r * _NHID:(r + 1) * _NHID] = jnp.dot(
                h1, w2_ref[r], preferred_element_type=jnp.float32
            ).astype(jnp.bfloat16)


def _layer2_body(adjb_ref, xw_ref, fc1w_ref, fc1b_ref, fc2w_ref, fc2b_ref,
                 out_ref):
    h2 = None
    for r in range(_SUPPORT):
        p = jnp.dot(
            adjb_ref[r], xw_ref[:, r * _NHID:(r + 1) * _NHID],
            preferred_element_type=jnp.float32,
        )
        h2 = p if h2 is None else h2 + p
    h2 = jnp.maximum(h2, 0.0)
    h3 = jnp.maximum(
        jnp.dot(h2, fc1w_ref[...], preferred_element_type=jnp.float32)
        + fc1b_ref[...],
        0.0,
    )
    logits = (
        jnp.dot(h3, fc2w_ref[...], preferred_element_type=jnp.float32)
        + fc2b_ref[...]
    )
    m = jnp.max(logits, axis=-1, keepdims=True)
    lse = m + jnp.log(jnp.sum(jnp.exp(logits - m), axis=-1, keepdims=True))
    out_ref[...] = logits - lse


def kernel(x, adj, basis1, comb1, basis2, comb2, fc1_w, fc1_b, fc2_w, fc2_b):
    W1 = jnp.einsum('rb,bio->rio', comb1, basis1)
    W2 = jnp.einsum('rb,bio->rio', comb2, basis2)
    fc2_wp = jnp.zeros((_NHID, _PADC), jnp.float32).at[:, :_NCLASS].set(fc2_w)
    fc2_bp = jnp.full((1, _PADC), -1e30, jnp.float32).at[0, :_NCLASS].set(fc2_b)
    fc1_b2 = fc1_b.reshape(1, _NHID)

    xw1 = pl.pallas_call(
        _xw_body,
        grid=(8,),
        in_specs=[
            pl.BlockSpec((_N // 8, _NHID), lambda i: (i, 0)),
            pl.BlockSpec((_SUPPORT, _NHID, _NHID), lambda i: (0, 0, 0)),
        ],
        out_specs=pl.BlockSpec((_N // 8, _XW), lambda i: (i, 0)),
        out_shape=jax.ShapeDtypeStruct((_N, _XW), jnp.bfloat16),
    )(x, W1)

    adjb, xw2 = pl.pallas_call(
        _layer1_body,
        grid=(_N // _TM, _N // _TK),
        in_specs=[
            pl.BlockSpec((_SUPPORT, _TM, _TK), lambda i, k: (0, i, k)),
            pl.BlockSpec((_N, _XW), lambda i, k: (0, 0)),
            pl.BlockSpec((_SUPPORT, _NHID, _NHID), lambda i, k: (0, 0, 0)),
        ],
        out_specs=[
            pl.BlockSpec((_SUPPORT, _TM, _TK), lambda i, k: (0, i, k)),
            pl.BlockSpec((_TM, _XW), lambda i, k: (i, 0)),
        ],
        out_shape=[
            jax.ShapeDtypeStruct((_SUPPORT, _N, _N), jnp.bfloat16),
            jax.ShapeDtypeStruct((_N, _XW), jnp.bfloat16),
        ],
        scratch_shapes=[pltpu.VMEM((_TM, _NHID), jnp.float32)],
        compiler_params=pltpu.CompilerParams(
            dimension_semantics=("parallel", "arbitrary"),
        ),
    )(adj, xw1, W2)

    logp = pl.pallas_call(
        _layer2_body,
        grid=(_N // _TM,),
        in_specs=[
            pl.BlockSpec((_SUPPORT, _TM, _N), lambda i: (0, i, 0)),
            pl.BlockSpec((_N, _XW), lambda i: (0, 0)),
            pl.BlockSpec((_NHID, _NHID), lambda i: (0, 0)),
            pl.BlockSpec((1, _NHID), lambda i: (0, 0)),
            pl.BlockSpec((_NHID, _PADC), lambda i: (0, 0)),
            pl.BlockSpec((1, _PADC), lambda i: (0, 0)),
        ],
        out_specs=pl.BlockSpec((_TM, _PADC), lambda i: (i, 0)),
        out_shape=jax.ShapeDtypeStruct((_N, _PADC), jnp.float32),
        compiler_params=pltpu.CompilerParams(
            dimension_semantics=("arbitrary",),
        ),
    )(adjb, xw2, fc1_w, fc1_b2, fc2_wp, fc2_bp)

    return logp[:, :_NCLASS]


# speedup vs baseline: 1.1615x; 1.1615x over previous
import jax
import jax.numpy as jnp
from jax.experimental import pallas as pl
from jax.experimental.pallas import tpu as pltpu

_N = 4096
_NHID = 512
_SUPPORT = 4
_NCLASS = 4
_PADC = 128

_TM = 512
_TK = 1024
_XW = _SUPPORT * _NHID


def _adj_matmul_step(adj_ref, xw_ref, acc_ref):
    a = adj_ref[...]
    xk = _TK * pl.program_id(1)
    part = None
    for r in range(_SUPPORT):
        p = jnp.dot(
            a[r].astype(jnp.bfloat16),
            xw_ref[pl.ds(xk, _TK), r * _NHID:(r + 1) * _NHID],
            preferred_element_type=jnp.float32,
        )
        part = p if part is None else part + p
    k = pl.program_id(1)

    @pl.when(k == 0)
    def _init():
        acc_ref[...] = part

    @pl.when(k > 0)
    def _accum():
        acc_ref[...] = acc_ref[...] + part


def _layer1_body(adj_ref, x_ref, w1_ref, w2_ref, xw2_ref, acc_ref, xw1_ref):
    i, k = pl.program_id(0), pl.program_id(1)

    @pl.when(i == 0)
    def _fill_xw1():
        xk = _TK * k
        xs = x_ref[pl.ds(xk, _TK), :]
        for r in range(_SUPPORT):
            xw1_ref[pl.ds(xk, _TK), r * _NHID:(r + 1) * _NHID] = jnp.dot(
                xs, w1_ref[r], preferred_element_type=jnp.float32
            ).astype(jnp.bfloat16)

    _adj_matmul_step(adj_ref, xw1_ref, acc_ref)

    @pl.when(k == pl.num_programs(1) - 1)
    def _epilogue():
        h1 = jnp.maximum(acc_ref[...], 0.0)
        for r in range(_SUPPORT):
            xw2_ref[:, r * _NHID:(r + 1) * _NHID] = jnp.dot(
                h1, w2_ref[r], preferred_element_type=jnp.float32
            ).astype(jnp.bfloat16)


def _layer2_body(adj_ref, xw2_ref, fc1w_ref, fc1b_ref, fc2w_ref, fc2b_ref,
                 out_ref, acc_ref):
    _adj_matmul_step(adj_ref, xw2_ref, acc_ref)

    @pl.when(pl.program_id(1) == pl.num_programs(1) - 1)
    def _epilogue():
        h2 = jnp.maximum(acc_ref[...], 0.0)
        h3 = jnp.maximum(
            jnp.dot(h2, fc1w_ref[...], preferred_element_type=jnp.float32)
            + fc1b_ref[...],
            0.0,
        )
        logits = (
            jnp.dot(h3, fc2w_ref[...], preferred_element_type=jnp.float32)
            + fc2b_ref[...]
        )
        m = jnp.max(logits, axis=-1, keepdims=True)
        lse = m + jnp.log(
            jnp.sum(jnp.exp(logits - m), axis=-1, keepdims=True)
        )
        out_ref[...] = logits - lse


def kernel(x, adj, basis1, comb1, basis2, comb2, fc1_w, fc1_b, fc2_w, fc2_b):
    W1 = jnp.einsum('rb,bio->rio', comb1, basis1)
    W2 = jnp.einsum('rb,bio->rio', comb2, basis2)
    fc2_wp = jnp.zeros((_NHID, _PADC), jnp.float32).at[:, :_NCLASS].set(fc2_w)
    fc2_bp = jnp.full((1, _PADC), -1e30, jnp.float32).at[0, :_NCLASS].set(fc2_b)
    fc1_b2 = fc1_b.reshape(1, _NHID)

    grid = (_N // _TM, _N // _TK)
    adj_spec = pl.BlockSpec((_SUPPORT, _TM, _TK), lambda i, k: (0, i, k))
    cparams = pltpu.CompilerParams(
        dimension_semantics=("parallel", "arbitrary"),
        vmem_limit_bytes=100 * 1024 * 1024,
    )

    xw2 = pl.pallas_call(
        _layer1_body,
        grid=grid,
        in_specs=[
            adj_spec,
            pl.BlockSpec((_N, _NHID), lambda i, k: (0, 0)),
            pl.BlockSpec((_SUPPORT, _NHID, _NHID), lambda i, k: (0, 0, 0)),
            pl.BlockSpec((_SUPPORT, _NHID, _NHID), lambda i, k: (0, 0, 0)),
        ],
        out_specs=pl.BlockSpec((_TM, _XW), lambda i, k: (i, 0)),
        out_shape=jax.ShapeDtypeStruct((_N, _XW), jnp.bfloat16),
        scratch_shapes=[
            pltpu.VMEM((_TM, _NHID), jnp.float32),
            pltpu.VMEM((_N, _XW), jnp.bfloat16),
        ],
        compiler_params=cparams,
    )(adj, x, W1, W2)

    logp = pl.pallas_call(
        _layer2_body,
        grid=grid,
        in_specs=[
            adj_spec,
            pl.BlockSpec((_N, _XW), lambda i, k: (0, 0)),
            pl.BlockSpec((_NHID, _NHID), lambda i, k: (0, 0)),
            pl.BlockSpec((1, _NHID), lambda i, k: (0, 0)),
            pl.BlockSpec((_NHID, _PADC), lambda i, k: (0, 0)),
            pl.BlockSpec((1, _PADC), lambda i, k: (0, 0)),
        ],
        out_specs=pl.BlockSpec((_TM, _PADC), lambda i, k: (i, 0)),
        out_shape=jax.ShapeDtypeStruct((_N, _PADC), jnp.float32),
        scratch_shapes=[pltpu.VMEM((_TM, _NHID), jnp.float32)],
        compiler_params=cparams,
    )(adj, xw2, fc1_w, fc1_b2, fc2_wp, fc2_bp)

    return logp[:, :_NCLASS]


# speedup vs baseline: 1.1634x; 1.0016x over previous
import jax
import jax.numpy as jnp
from jax.experimental import pallas as pl
from jax.experimental.pallas import tpu as pltpu

_N = 4096
_NHID = 512
_SUPPORT = 4
_NCLASS = 4
_PADC = 128

_TM = 512
_TK = 1024
_XW = _SUPPORT * _NHID


def _adj_matmul_step(adj_ref, xw_ref, acc_ref):
    a = adj_ref[...]
    xk = _TK * pl.program_id(1)
    part = None
    for r in range(_SUPPORT):
        p = jnp.dot(
            a[r],
            xw_ref[pl.ds(xk, _TK), r * _NHID:(r + 1) * _NHID].astype(
                jnp.float32),
            preferred_element_type=jnp.float32,
        )
        part = p if part is None else part + p
    k = pl.program_id(1)

    @pl.when(k == 0)
    def _init():
        acc_ref[...] = part

    @pl.when(k > 0)
    def _accum():
        acc_ref[...] = acc_ref[...] + part


def _layer1_body(adj_ref, x_ref, w1_ref, w2_ref, xw2_ref, acc_ref, xw1_ref):
    i, k = pl.program_id(0), pl.program_id(1)

    @pl.when(i == 0)
    def _fill_xw1():
        xk = _TK * k
        xs = x_ref[pl.ds(xk, _TK), :]
        for r in range(_SUPPORT):
            xw1_ref[pl.ds(xk, _TK), r * _NHID:(r + 1) * _NHID] = jnp.dot(
                xs, w1_ref[r], preferred_element_type=jnp.float32
            ).astype(jnp.bfloat16)

    _adj_matmul_step(adj_ref, xw1_ref, acc_ref)

    @pl.when(k == pl.num_programs(1) - 1)
    def _epilogue():
        h1 = jnp.maximum(acc_ref[...], 0.0)
        for r in range(_SUPPORT):
            xw2_ref[:, r * _NHID:(r + 1) * _NHID] = jnp.dot(
                h1, w2_ref[r], preferred_element_type=jnp.float32
            ).astype(jnp.bfloat16)


def _layer2_body(adj_ref, xw2_ref, fc1w_ref, fc1b_ref, fc2w_ref, fc2b_ref,
                 out_ref, acc_ref):
    _adj_matmul_step(adj_ref, xw2_ref, acc_ref)

    @pl.when(pl.program_id(1) == pl.num_programs(1) - 1)
    def _epilogue():
        h2 = jnp.maximum(acc_ref[...], 0.0)
        h3 = jnp.maximum(
            jnp.dot(h2, fc1w_ref[...], preferred_element_type=jnp.float32)
            + fc1b_ref[...],
            0.0,
        )
        logits = (
            jnp.dot(h3, fc2w_ref[...], preferred_element_type=jnp.float32)
            + fc2b_ref[...]
        )
        m = jnp.max(logits, axis=-1, keepdims=True)
        lse = m + jnp.log(
            jnp.sum(jnp.exp(logits - m), axis=-1, keepdims=True)
        )
        out_ref[...] = logits - lse


def kernel(x, adj, basis1, comb1, basis2, comb2, fc1_w, fc1_b, fc2_w, fc2_b):
    W1 = jnp.einsum('rb,bio->rio', comb1, basis1)
    W2 = jnp.einsum('rb,bio->rio', comb2, basis2)
    fc2_wp = jnp.zeros((_NHID, _PADC), jnp.float32).at[:, :_NCLASS].set(fc2_w)
    fc2_bp = jnp.full((1, _PADC), -1e30, jnp.float32).at[0, :_NCLASS].set(fc2_b)
    fc1_b2 = fc1_b.reshape(1, _NHID)

    grid = (_N // _TM, _N // _TK)
    adj_spec = pl.BlockSpec((_SUPPORT, _TM, _TK), lambda i, k: (0, i, k))
    cparams = pltpu.CompilerParams(
        dimension_semantics=("parallel", "arbitrary"),
        vmem_limit_bytes=100 * 1024 * 1024,
    )

    xw2 = pl.pallas_call(
        _layer1_body,
        grid=grid,
        in_specs=[
            adj_spec,
            pl.BlockSpec((_N, _NHID), lambda i, k: (0, 0)),
            pl.BlockSpec((_SUPPORT, _NHID, _NHID), lambda i, k: (0, 0, 0)),
            pl.BlockSpec((_SUPPORT, _NHID, _NHID), lambda i, k: (0, 0, 0)),
        ],
        out_specs=pl.BlockSpec((_TM, _XW), lambda i, k: (i, 0)),
        out_shape=jax.ShapeDtypeStruct((_N, _XW), jnp.bfloat16),
        scratch_shapes=[
            pltpu.VMEM((_TM, _NHID), jnp.float32),
            pltpu.VMEM((_N, _XW), jnp.bfloat16),
        ],
        compiler_params=cparams,
    )(adj, x, W1, W2)

    logp = pl.pallas_call(
        _layer2_body,
        grid=grid,
        in_specs=[
            adj_spec,
            pl.BlockSpec((_N, _XW), lambda i, k: (0, 0)),
            pl.BlockSpec((_NHID, _NHID), lambda i, k: (0, 0)),
            pl.BlockSpec((1, _NHID), lambda i, k: (0, 0)),
            pl.BlockSpec((_NHID, _PADC), lambda i, k: (0, 0)),
            pl.BlockSpec((1, _PADC), lambda i, k: (0, 0)),
        ],
        out_specs=pl.BlockSpec((_TM, _PADC), lambda i, k: (i, 0)),
        out_shape=jax.ShapeDtypeStruct((_N, _PADC), jnp.float32),
        scratch_shapes=[pltpu.VMEM((_TM, _NHID), jnp.float32)],
        compiler_params=cparams,
    )(adj, xw2, fc1_w, fc1_b2, fc2_wp, fc2_bp)

    return logp[:, :_NCLASS]


# speedup vs baseline: 1.1638x; 1.0003x over previous
import jax
import jax.numpy as jnp
from jax.experimental import pallas as pl
from jax.experimental.pallas import tpu as pltpu

_N = 4096
_NHID = 512
_SUPPORT = 4
_NCLASS = 4
_PADC = 128

_TM = 256
_TK = 2048
_XW = _SUPPORT * _NHID


def _xw_body(x_ref, w_ref, out_ref):
    for r in range(_SUPPORT):
        out_ref[:, r * _NHID:(r + 1) * _NHID] = jnp.dot(
            x_ref[...], w_ref[r], preferred_element_type=jnp.float32
        ).astype(jnp.bfloat16)


def _adj_matmul_step(adj_ref, xw_ref, acc_ref):
    a = adj_ref[...]
    xk = _TK * pl.program_id(2)
    part = None
    for r in range(_SUPPORT):
        p = jnp.dot(
            a[r],
            xw_ref[pl.ds(xk, _TK), r * _NHID:(r + 1) * _NHID].astype(
                jnp.float32),
            preferred_element_type=jnp.float32,
        )
        part = p if part is None else part + p
    k = pl.program_id(2)

    @pl.when(k == 0)
    def _init():
        acc_ref[...] = part

    @pl.when(k > 0)
    def _accum():
        acc_ref[...] = acc_ref[...] + part


def _main_body(adj_ref, xw1_ref, w2_ref, fc1w_ref, fc1b_ref, fc2w_ref,
               fc2b_ref, out_ref, acc_ref, xw2_ref):
    p, i, k = pl.program_id(0), pl.program_id(1), pl.program_id(2)
    nk = pl.num_programs(2)

    @pl.when(p == 0)
    def _layer1():
        _adj_matmul_step(adj_ref, xw1_ref, acc_ref)

        @pl.when(k == nk - 1)
        def _epi1():
            h1 = jnp.maximum(acc_ref[...], 0.0)
            im = _TM * i
            for r in range(_SUPPORT):
                xw2_ref[pl.ds(im, _TM), r * _NHID:(r + 1) * _NHID] = jnp.dot(
                    h1, w2_ref[r], preferred_element_type=jnp.float32
                ).astype(jnp.bfloat16)

    @pl.when(p == 1)
    def _layer2():
        _adj_matmul_step(adj_ref, xw2_ref, acc_ref)

        @pl.when(k == nk - 1)
        def _epi2():
            h2 = jnp.maximum(acc_ref[...], 0.0)
            h3 = jnp.maximum(
                jnp.dot(h2, fc1w_ref[...], preferred_element_type=jnp.float32)
                + fc1b_ref[...],
                0.0,
            )
            logits = (
                jnp.dot(h3, fc2w_ref[...],
                        preferred_element_type=jnp.float32)
                + fc2b_ref[...]
            )
            m = jnp.max(logits, axis=-1, keepdims=True)
            lse = m + jnp.log(
                jnp.sum(jnp.exp(logits - m), axis=-1, keepdims=True)
            )
            out_ref[...] = logits - lse


def kernel(x, adj, basis1, comb1, basis2, comb2, fc1_w, fc1_b, fc2_w, fc2_b):
    W1 = jnp.einsum('rb,bio->rio', comb1, basis1)
    W2 = jnp.einsum('rb,bio->rio', comb2, basis2)
    fc2_wp = jnp.zeros((_NHID, _PADC), jnp.float32).at[:, :_NCLASS].set(fc2_w)
    fc2_bp = jnp.full((1, _PADC), -1e30, jnp.float32).at[0, :_NCLASS].set(fc2_b)
    fc1_b2 = fc1_b.reshape(1, _NHID)

    xw1 = pl.pallas_call(
        _xw_body,
        grid=(8,),
        in_specs=[
            pl.BlockSpec((_N // 8, _NHID), lambda i: (i, 0)),
            pl.BlockSpec((_SUPPORT, _NHID, _NHID), lambda i: (0, 0, 0)),
        ],
        out_specs=pl.BlockSpec((_N // 8, _XW), lambda i: (i, 0)),
        out_shape=jax.ShapeDtypeStruct((_N, _XW), jnp.bfloat16),
    )(x, W1)

    logp = pl.pallas_call(
        _main_body,
        grid=(2, _N // _TM, _N // _TK),
        in_specs=[
            pl.BlockSpec((_SUPPORT, _TM, _TK), lambda p, i, k: (0, i, k)),
            pl.BlockSpec((_N, _XW), lambda p, i, k: (0, 0)),
            pl.BlockSpec((_SUPPORT, _NHID, _NHID),
                         lambda p, i, k: (0, 0, 0)),
            pl.BlockSpec((_NHID, _NHID), lambda p, i, k: (0, 0)),
            pl.BlockSpec((1, _NHID), lambda p, i, k: (0, 0)),
            pl.BlockSpec((_NHID, _PADC), lambda p, i, k: (0, 0)),
            pl.BlockSpec((1, _PADC), lambda p, i, k: (0, 0)),
        ],
        out_specs=pl.BlockSpec((_TM, _PADC), lambda p, i, k: (i * p, 0)),
        out_shape=jax.ShapeDtypeStruct((_N, _PADC), jnp.float32),
        scratch_shapes=[
            pltpu.VMEM((_TM, _NHID), jnp.float32),
            pltpu.VMEM((_N, _XW), jnp.bfloat16),
        ],
        compiler_params=pltpu.CompilerParams(
            dimension_semantics=("arbitrary", "arbitrary", "arbitrary"),
            vmem_limit_bytes=100 * 1024 * 1024,
        ),
    )(adj, xw1, W2, fc1_w, fc1_b2, fc2_wp, fc2_bp)

    return logp[:, :_NCLASS]
